# Initial kernel scaffold; baseline (speedup 1.0000x reference)
import jax
import jax.numpy as jnp
from jax.experimental import pallas as pl


def kernel(im_q, im_k, W_F, b_F, W_C, b_C, mW_F, mb_F, mW_C, mb_C, features_mem):
    raise NotImplementedError("write your pallas kernel here")


# speedup vs baseline: 1.2803x; 1.2803x over previous
import functools

import jax
import jax.numpy as jnp
from jax.experimental import pallas as pl


_M = 0.999
_T_MOCO = 0.07


def _stage_a_body(imq_ref, imk_ref, wf_ref, bf_ref, wc_ref, bc_ref,
                  mwf_ref, mbf_ref,
                  feats_ref, logq_ref, q_ref, k_ref, lpos_ref):
    feats = jnp.dot(imq_ref[...], wf_ref[...],
                    preferred_element_type=jnp.float32) + bf_ref[...]
    feats_ref[...] = feats
    logq_ref[...] = jnp.dot(feats, wc_ref[...],
                            preferred_element_type=jnp.float32) + bc_ref[...]
    qn = jnp.sqrt(jnp.sum(feats * feats, axis=1, keepdims=True))
    q = feats / jnp.maximum(qn, 1e-12)
    q_ref[...] = q
    mw_new = mwf_ref[...] * _M + wf_ref[...] * (1.0 - _M)
    mb_new = mbf_ref[...] * _M + bf_ref[...] * (1.0 - _M)
    kf = jnp.dot(imk_ref[...], mw_new,
                 preferred_element_type=jnp.float32) + mb_new
    kn = jnp.sqrt(jnp.sum(kf * kf, axis=1, keepdims=True))
    kv = kf / jnp.maximum(kn, 1e-12)
    k_ref[...] = kv
    lpos_ref[...] = jnp.sum(q * kv, axis=1, keepdims=True)


def _stage_b_body(q_ref, lpos_ref, fp_ref, out_ref):
    j = pl.program_id(0)
    inv_t = 1.0 / _T_MOCO
    out_ref[...] = jnp.dot(q_ref[...], fp_ref[...],
                           preferred_element_type=jnp.float32) * inv_t

    @pl.when(j == 0)
    def _():
        out_ref[:, 0:1] = lpos_ref[...] * inv_t


@jax.jit
def kernel(im_q, im_k, W_F, b_F, W_C, b_C, mW_F, mb_F, mW_C, mb_C,
           features_mem):
    B, D = im_q.shape
    C = W_F.shape[1]
    NC = W_C.shape[1]
    K = features_mem.shape[1]
    KP = K + 1

    bf2 = b_F.reshape(1, C)
    bc2 = b_C.reshape(1, NC)
    mbf2 = mb_F.reshape(1, C)

    BR = 256
    feats_q, logits_q, q, k, l_pos = pl.pallas_call(
        _stage_a_body,
        grid=(B // BR,),
        in_specs=[
            pl.BlockSpec((BR, D), lambda i: (i, 0)),
            pl.BlockSpec((BR, D), lambda i: (i, 0)),
            pl.BlockSpec((D, C), lambda i: (0, 0)),
            pl.BlockSpec((1, C), lambda i: (0, 0)),
            pl.BlockSpec((C, NC), lambda i: (0, 0)),
            pl.BlockSpec((1, NC), lambda i: (0, 0)),
            pl.BlockSpec((D, C), lambda i: (0, 0)),
            pl.BlockSpec((1, C), lambda i: (0, 0)),
        ],
        out_specs=[
            pl.BlockSpec((BR, C), lambda i: (i, 0)),
            pl.BlockSpec((BR, NC), lambda i: (i, 0)),
            pl.BlockSpec((BR, C), lambda i: (i, 0)),
            pl.BlockSpec((BR, C), lambda i: (i, 0)),
            pl.BlockSpec((BR, 1), lambda i: (i, 0)),
        ],
        out_shape=[
            jax.ShapeDtypeStruct((B, C), jnp.float32),
            jax.ShapeDtypeStruct((B, NC), jnp.float32),
            jax.ShapeDtypeStruct((B, C), jnp.float32),
            jax.ShapeDtypeStruct((B, C), jnp.float32),
            jax.ShapeDtypeStruct((B, 1), jnp.float32),
        ],
    )(im_q, im_k, W_F, bf2, W_C, bc2, mW_F, mbf2)

    fp = jnp.concatenate([jnp.zeros((C, 1), jnp.float32), features_mem],
                         axis=1)

    W = 2048
    logits_ins = pl.pallas_call(
        _stage_b_body,
        grid=(pl.cdiv(KP, W),),
        in_specs=[
            pl.BlockSpec((B, C), lambda j: (0, 0)),
            pl.BlockSpec((B, 1), lambda j: (0, 0)),
            pl.BlockSpec((C, W), lambda j: (0, j)),
        ],
        out_specs=pl.BlockSpec((B, W), lambda j: (0, j)),
        out_shape=jax.ShapeDtypeStruct((B, KP), jnp.float32),
    )(q, l_pos, fp)

    return (feats_q, logits_q, logits_ins, k)


# speedup vs baseline: 1.2952x; 1.0116x over previous
import functools

import jax
import jax.numpy as jnp
from jax.experimental import pallas as pl


_M = 0.999
_T_MOCO = 0.07


def _bdot(a, b):
    return jnp.dot(a.astype(jnp.bfloat16), b.astype(jnp.bfloat16),
                   preferred_element_type=jnp.float32)


def _stage_a_body(imq_ref, imk_ref, wf_ref, bf_ref, wc_ref, bc_ref,
                  mwf_ref, mbf_ref,
                  feats_ref, logq_ref, q_ref, k_ref, lpos_ref):
    feats = _bdot(imq_ref[...], wf_ref[...]) + bf_ref[...]
    feats_ref[...] = feats
    logq_ref[...] = _bdot(feats, wc_ref[...]) + bc_ref[...]
    qn = jnp.sqrt(jnp.sum(feats * feats, axis=1, keepdims=True))
    q = feats / jnp.maximum(qn, 1e-12)
    q_ref[...] = q
    mw_new = mwf_ref[...] * _M + wf_ref[...] * (1.0 - _M)
    mb_new = mbf_ref[...] * _M + bf_ref[...] * (1.0 - _M)
    kf = _bdot(imk_ref[...], mw_new) + mb_new
    kn = jnp.sqrt(jnp.sum(kf * kf, axis=1, keepdims=True))
    kv = kf / jnp.maximum(kn, 1e-12)
    k_ref[...] = kv
    lpos_ref[...] = jnp.sum(q * kv, axis=1, keepdims=True)


def _stage_b_body(q_ref, lpos_ref, fp_ref, out_ref):
    j = pl.program_id(0)
    inv_t = 1.0 / _T_MOCO
    out_ref[...] = _bdot(q_ref[...], fp_ref[...]) * inv_t

    @pl.when(j == 0)
    def _():
        out_ref[:, 0:1] = lpos_ref[...] * inv_t


@jax.jit
def kernel(im_q, im_k, W_F, b_F, W_C, b_C, mW_F, mb_F, mW_C, mb_C,
           features_mem):
    B, D = im_q.shape
    C = W_F.shape[1]
    NC = W_C.shape[1]
    K = features_mem.shape[1]
    KP = K + 1

    bf2 = b_F.reshape(1, C)
    bc2 = b_C.reshape(1, NC)
    mbf2 = mb_F.reshape(1, C)

    BR = 256
    feats_q, logits_q, q, k, l_pos = pl.pallas_call(
        _stage_a_body,
        grid=(B // BR,),
        in_specs=[
            pl.BlockSpec((BR, D), lambda i: (i, 0)),
            pl.BlockSpec((BR, D), lambda i: (i, 0)),
            pl.BlockSpec((D, C), lambda i: (0, 0)),
            pl.BlockSpec((1, C), lambda i: (0, 0)),
            pl.BlockSpec((C, NC), lambda i: (0, 0)),
            pl.BlockSpec((1, NC), lambda i: (0, 0)),
            pl.BlockSpec((D, C), lambda i: (0, 0)),
            pl.BlockSpec((1, C), lambda i: (0, 0)),
        ],
        out_specs=[
            pl.BlockSpec((BR, C), lambda i: (i, 0)),
            pl.BlockSpec((BR, NC), lambda i: (i, 0)),
            pl.BlockSpec((BR, C), lambda i: (i, 0)),
            pl.BlockSpec((BR, C), lambda i: (i, 0)),
            pl.BlockSpec((BR, 1), lambda i: (i, 0)),
        ],
        out_shape=[
            jax.ShapeDtypeStruct((B, C), jnp.float32),
            jax.ShapeDtypeStruct((B, NC), jnp.float32),
            jax.ShapeDtypeStruct((B, C), jnp.float32),
            jax.ShapeDtypeStruct((B, C), jnp.float32),
            jax.ShapeDtypeStruct((B, 1), jnp.float32),
        ],
    )(im_q, im_k, W_F, bf2, W_C, bc2, mW_F, mbf2)

    fp = jnp.concatenate([jnp.zeros((C, 1), jnp.float32), features_mem],
                         axis=1)

    W = 2048
    logits_ins = pl.pallas_call(
        _stage_b_body,
        grid=(pl.cdiv(KP, W),),
        in_specs=[
            pl.BlockSpec((B, C), lambda j: (0, 0)),
            pl.BlockSpec((B, 1), lambda j: (0, 0)),
            pl.BlockSpec((C, W), lambda j: (0, j)),
        ],
        out_specs=pl.BlockSpec((B, W), lambda j: (0, j)),
        out_shape=jax.ShapeDtypeStruct((B, KP), jnp.float32),
    )(q, l_pos, fp)

    return (feats_q, logits_q, logits_ins, k)


# speedup vs baseline: 1.3677x; 1.0560x over previous
import jax
import jax.numpy as jnp
from jax.experimental import pallas as pl


_M = 0.999
_T_MOCO = 0.07


def _bdot(a, b):
    return jnp.dot(a.astype(jnp.bfloat16), b.astype(jnp.bfloat16),
                   preferred_element_type=jnp.float32)


def _body(imq_ref, imk_ref, wf_ref, bf_ref, wc_ref, bc_ref, mwf_ref, mbf_ref,
          fp_ref, feats_ref, logq_ref, ins_ref, k_ref):
    inv_t = 1.0 / _T_MOCO
    feats = _bdot(imq_ref[...], wf_ref[...]) + bf_ref[...]
    feats_ref[...] = feats
    logq_ref[...] = _bdot(feats, wc_ref[...]) + bc_ref[...]
    qn = jnp.sqrt(jnp.sum(feats * feats, axis=1, keepdims=True))
    q = feats / jnp.maximum(qn, 1e-12)

    mw_new = mwf_ref[...] * _M + wf_ref[...] * (1.0 - _M)
    mb_new = mbf_ref[...] * _M + bf_ref[...] * (1.0 - _M)
    kf = _bdot(imk_ref[...], mw_new) + mb_new
    kn = jnp.sqrt(jnp.sum(kf * kf, axis=1, keepdims=True))
    kv = kf / jnp.maximum(kn, 1e-12)
    k_ref[...] = kv

    qs = (q * inv_t).astype(jnp.bfloat16)
    ins_ref[...] = jnp.dot(qs, fp_ref[...],
                           preferred_element_type=jnp.float32)
    ins_ref[:, 0:1] = jnp.sum(q * kv, axis=1, keepdims=True) * inv_t


@jax.jit
def kernel(im_q, im_k, W_F, b_F, W_C, b_C, mW_F, mb_F, mW_C, mb_C,
           features_mem):
    B, D = im_q.shape
    C = W_F.shape[1]
    NC = W_C.shape[1]
    K = features_mem.shape[1]
    KP = K + 1

    bf2 = b_F.reshape(1, C)
    bc2 = b_C.reshape(1, NC)
    mbf2 = mb_F.reshape(1, C)

    fp = jnp.concatenate(
        [jnp.zeros((C, 1), jnp.bfloat16), features_mem.astype(jnp.bfloat16)],
        axis=1)

    BR = 128
    feats_q, logits_q, logits_ins, k = pl.pallas_call(
        _body,
        grid=(B // BR,),
        in_specs=[
            pl.BlockSpec((BR, D), lambda i: (i, 0)),
            pl.BlockSpec((BR, D), lambda i: (i, 0)),
            pl.BlockSpec((D, C), lambda i: (0, 0)),
            pl.BlockSpec((1, C), lambda i: (0, 0)),
            pl.BlockSpec((C, NC), lambda i: (0, 0)),
            pl.BlockSpec((1, NC), lambda i: (0, 0)),
            pl.BlockSpec((D, C), lambda i: (0, 0)),
            pl.BlockSpec((1, C), lambda i: (0, 0)),
            pl.BlockSpec((C, KP), lambda i: (0, 0)),
        ],
        out_specs=[
            pl.BlockSpec((BR, C), lambda i: (i, 0)),
            pl.BlockSpec((BR, NC), lambda i: (i, 0)),
            pl.BlockSpec((BR, KP), lambda i: (i, 0)),
            pl.BlockSpec((BR, C), lambda i: (i, 0)),
        ],
        out_shape=[
            jax.ShapeDtypeStruct((B, C), jnp.float32),
            jax.ShapeDtypeStruct((B, NC), jnp.float32),
            jax.ShapeDtypeStruct((B, KP), jnp.float32),
            jax.ShapeDtypeStruct((B, C), jnp.float32),
        ],
    )(im_q, im_k, W_F, bf2, W_C, bc2, mW_F, mbf2, fp)

    return (feats_q, logits_q, logits_ins, k)
